# Initial kernel scaffold; baseline (speedup 1.0000x reference)
import jax
import jax.numpy as jnp
from jax.experimental import pallas as pl


def kernel(x, edge_index, W1l, b1, W1r, W2l, b2, W2r):
    raise NotImplementedError("write your pallas kernel here")


# speedup vs baseline: 11.6206x; 11.6206x over previous
import functools

import jax
import jax.numpy as jnp
from jax import lax
from jax.experimental import pallas as pl
from jax.experimental.pallas import tpu as pltpu
from jax.experimental.pallas import tpu_sc as plsc

N = 10000
D = 128
H = 16
E = 320000

NC = 2
NS = 16
NW = NC * NS
CHUNK = 128
CPW = 79
EPW = CPW * CHUNK
EP = NW * EPW

NP = 10240
RPS = NP // NS


def _seg_body(compute_cnt, src_hbm, dst_hbm, y_hbm, *rest):
    if compute_cnt:
        part_out, cnt_out, src_v, dst_v, rows_v, ones_v, zrow_v, zcnt_v, \
            acc_sh, cnt_sh, sem = rest
    else:
        part_out, src_v, dst_v, rows_v, zrow_v, acc_sh, sem = rest

    c = lax.axis_index("c")
    s = lax.axis_index("s")
    wid = s * NC + c

    pltpu.sync_copy(src_hbm.at[wid], src_v)
    pltpu.sync_copy(dst_hbm.at[wid], dst_v)

    def fill_zrow(i, _):
        zrow_v[i] = jnp.zeros((16,), jnp.float32)
        return 0
    lax.fori_loop(0, CHUNK, fill_zrow, 0)
    if compute_cnt:
        def fill_ones(i, _):
            ones_v[pl.ds(i * 16, 16)] = jnp.ones((16,), jnp.float32)
            zcnt_v[pl.ds(i * 16, 16)] = jnp.zeros((16,), jnp.float32)
            return 0
        lax.fori_loop(0, CHUNK // 16, fill_ones, 0)

    def zero_acc(k, _):
        pltpu.sync_copy(zrow_v, acc_sh.at[pl.ds(s * RPS + k * CHUNK, CHUNK)])
        if compute_cnt:
            pltpu.sync_copy(zcnt_v, cnt_sh.at[pl.ds(s * RPS + k * CHUNK, CHUNK)])
        return 0
    lax.fori_loop(0, RPS // CHUNK, zero_acc, 0)
    plsc.subcore_barrier()

    def chunk_body(j, _):
        pltpu.async_copy(y_hbm.at[src_v.at[j]], rows_v, sem).wait()
        pltpu.sync_copy(rows_v, acc_sh.at[dst_v.at[j]], add=True)
        if compute_cnt:
            pltpu.sync_copy(ones_v, cnt_sh.at[dst_v.at[j]], add=True)
        return 0
    lax.fori_loop(0, CPW, chunk_body, 0)
    plsc.subcore_barrier()

    pltpu.sync_copy(acc_sh.at[pl.ds(s * RPS, RPS)],
                    part_out.at[c, pl.ds(s * RPS, RPS)])
    if compute_cnt:
        pltpu.sync_copy(cnt_sh.at[pl.ds(s * RPS, RPS)],
                        cnt_out.at[c, pl.ds(s * RPS, RPS)])


def _make_sc_segment(compute_cnt):
    mesh = plsc.VectorSubcoreMesh(core_axis_name="c", subcore_axis_name="s")
    out_type = [jax.ShapeDtypeStruct((NC, NP, H), jnp.float32)]
    scratch = [
        pltpu.VMEM((CPW, CHUNK), jnp.int32),
        pltpu.VMEM((CPW, CHUNK), jnp.int32),
        pltpu.VMEM((CHUNK, H), jnp.float32),
    ]
    if compute_cnt:
        out_type.append(jax.ShapeDtypeStruct((NC, NP), jnp.float32))
        scratch.append(pltpu.VMEM((CHUNK,), jnp.float32))
    scratch.append(pltpu.VMEM((CHUNK, H), jnp.float32))
    if compute_cnt:
        scratch.append(pltpu.VMEM((CHUNK,), jnp.float32))
    scratch.append(pltpu.VMEM_SHARED((NP, H), jnp.float32))
    if compute_cnt:
        scratch.append(pltpu.VMEM_SHARED((NP,), jnp.float32))
    scratch.append(pltpu.SemaphoreType.DMA)
    return pl.kernel(
        functools.partial(_seg_body, compute_cnt),
        out_type=tuple(out_type),
        mesh=mesh,
        scratch_types=tuple(scratch),
        compiler_params=pltpu.CompilerParams(use_tc_tiling_on_sc=False),
    )


def _tc_pre(x_p, W1l, W1r):
    def body(x_ref, wl_ref, wr_ref, y_ref, xr_ref):
        xb = x_ref[...]
        dn = (((1,), (1,)), ((), ()))
        y_ref[...] = lax.dot_general(xb, wl_ref[...], dn,
                                     preferred_element_type=jnp.float32)
        xr_ref[...] = lax.dot_general(xb, wr_ref[...], dn,
                                      preferred_element_type=jnp.float32)
    return pl.pallas_call(
        body,
        out_shape=(jax.ShapeDtypeStruct((NP, H), jnp.float32),
                   jax.ShapeDtypeStruct((NP, H), jnp.float32)),
    )(x_p, W1l, W1r)


def _tc_mid(part, cntp, xr1, b1, W2l, W2r):
    def body(part_ref, cnt_ref, xr_ref, b1_ref, wl_ref, wr_ref,
             y2_ref, hr2_ref):
        seg = part_ref[0] + part_ref[1]
        cnt = jnp.clip(cnt_ref[0] + cnt_ref[1], 1.0, None)
        h = jax.nn.sigmoid(seg / cnt + b1_ref[...] + xr_ref[...])
        dn = (((1,), (1,)), ((), ()))
        y2_ref[...] = lax.dot_general(h, wl_ref[...], dn,
                                      preferred_element_type=jnp.float32)
        hr2_ref[...] = lax.dot_general(h, wr_ref[...], dn,
                                       preferred_element_type=jnp.float32)
    return pl.pallas_call(
        body,
        out_shape=(jax.ShapeDtypeStruct((NP, H), jnp.float32),
                   jax.ShapeDtypeStruct((NP, H), jnp.float32)),
    )(part, cntp, xr1, b1, W2l, W2r)


def _tc_final(part, cntp, hr2, b2):
    def body(part_ref, cnt_ref, hr_ref, b2_ref, out_ref):
        seg = part_ref[0] + part_ref[1]
        cnt = jnp.clip(cnt_ref[0] + cnt_ref[1], 1.0, None)
        z = seg / cnt + b2_ref[...] + hr_ref[...]
        m = jnp.max(z, axis=1, keepdims=True)
        lse = jnp.log(jnp.sum(jnp.exp(z - m), axis=1, keepdims=True)) + m
        out_ref[...] = z - lse
    return pl.pallas_call(
        body,
        out_shape=jax.ShapeDtypeStruct((NP, 16), jnp.float32),
    )(part, cntp, hr2, b2)


@jax.jit
def kernel(x, edge_index, W1l, b1, W1r, W2l, b2, W2r):
    src = edge_index[0].astype(jnp.int32)
    dst = edge_index[1].astype(jnp.int32)
    pad = EP - E
    src_p = jnp.concatenate([src, jnp.zeros((pad,), jnp.int32)])
    dst_p = jnp.concatenate([dst, jnp.full((pad,), N, jnp.int32)])
    src_p = src_p.reshape(NW, CPW, CHUNK)
    dst_p = dst_p.reshape(NW, CPW, CHUNK)
    x_p = jnp.pad(x, ((0, NP - N), (0, 0)))

    y1, xr1 = _tc_pre(x_p, W1l, W1r)
    part1, cnt = _make_sc_segment(True)(src_p, dst_p, y1)
    cntp = cnt.reshape(NC, NP, 1)
    y2, hr2 = _tc_mid(part1, cntp, xr1, b1.reshape(1, H), W2l, W2r)
    (part2,) = _make_sc_segment(False)(src_p, dst_p, y2)
    out = _tc_final(part2, cntp, hr2, b2.reshape(1, 16))
    return out[:N]


# speedup vs baseline: 14.5718x; 1.2540x over previous
import functools

import jax
import jax.numpy as jnp
from jax import lax
from jax.experimental import pallas as pl
from jax.experimental.pallas import tpu as pltpu
from jax.experimental.pallas import tpu_sc as plsc

N = 10000
D = 128
H = 16
E = 320000

NC = 2
NS = 16
NW = NC * NS
CHUNK = 128
CPW = 80
EPW = CPW * CHUNK
EP = NW * EPW
NBUF = 8

NP = 10240
RPS = NP // NS


def _seg_body(compute_cnt, src_hbm, dst_hbm, y_hbm, *rest):
    if compute_cnt:
        part_out, cnt_out, src_v, dst_v, rows_v, ones_v, zrow_v, zcnt_v, \
            acc_sh, cnt_sh, gsem, ssem, csem = rest
    else:
        part_out, src_v, dst_v, rows_v, zrow_v, acc_sh, gsem, ssem = rest

    c = lax.axis_index("c")
    s = lax.axis_index("s")
    wid = s * NC + c

    pltpu.sync_copy(src_hbm.at[wid], src_v)
    pltpu.sync_copy(dst_hbm.at[wid], dst_v)

    def fill_zrow(i, _):
        zrow_v[i] = jnp.zeros((16,), jnp.float32)
        return 0
    lax.fori_loop(0, CHUNK, fill_zrow, 0)
    if compute_cnt:
        def fill_ones(i, _):
            ones_v[pl.ds(i * 16, 16)] = jnp.ones((16,), jnp.float32)
            zcnt_v[pl.ds(i * 16, 16)] = jnp.zeros((16,), jnp.float32)
            return 0
        lax.fori_loop(0, CHUNK // 16, fill_ones, 0)

    def zero_acc(k, _):
        pltpu.sync_copy(zrow_v, acc_sh.at[pl.ds(s * RPS + k * CHUNK, CHUNK)])
        if compute_cnt:
            pltpu.sync_copy(zcnt_v, cnt_sh.at[pl.ds(s * RPS + k * CHUNK, CHUNK)])
        return 0
    lax.fori_loop(0, RPS // CHUNK, zero_acc, 0)
    plsc.subcore_barrier()

    def gather(j, b):
        pltpu.async_copy(y_hbm.at[src_v.at[j]], rows_v.at[b], gsem.at[b])

    for b in range(4):
        gather(b, b)

    def group(g, _):
        for b in range(NBUF):
            j = g * NBUF + b
            pltpu.make_async_copy(
                y_hbm.at[src_v.at[j]], rows_v.at[b], gsem.at[b]).wait()
            pltpu.async_copy(
                rows_v.at[b], acc_sh.at[dst_v.at[j]], ssem.at[b], add=True)
            if compute_cnt:
                cb = b % 2

                @pl.when(j >= 2)
                def _():
                    pltpu.make_async_copy(
                        ones_v, cnt_sh.at[dst_v.at[j - 2]], csem.at[cb]).wait()
                pltpu.async_copy(
                    ones_v, cnt_sh.at[dst_v.at[j]], csem.at[cb], add=True)

            nb = (b + 4) % NBUF

            @pl.when(j + 4 < CPW)
            def _():
                @pl.when(j >= 4)
                def _():
                    pltpu.make_async_copy(
                        rows_v.at[nb], acc_sh.at[dst_v.at[j - 4]],
                        ssem.at[nb]).wait()
                gather(j + 4, nb)
        return 0
    lax.fori_loop(0, CPW // NBUF, group, 0)

    for b in range(NBUF):
        pltpu.make_async_copy(
            rows_v.at[b], acc_sh.at[dst_v.at[CPW - NBUF + b]],
            ssem.at[b]).wait()
    if compute_cnt:
        pltpu.make_async_copy(
            ones_v, cnt_sh.at[dst_v.at[CPW - 2]], csem.at[0]).wait()
        pltpu.make_async_copy(
            ones_v, cnt_sh.at[dst_v.at[CPW - 1]], csem.at[1]).wait()
    plsc.subcore_barrier()

    pltpu.sync_copy(acc_sh.at[pl.ds(s * RPS, RPS)],
                    part_out.at[c, pl.ds(s * RPS, RPS)])
    if compute_cnt:
        pltpu.sync_copy(cnt_sh.at[pl.ds(s * RPS, RPS)],
                        cnt_out.at[c, pl.ds(s * RPS, RPS)])


def _make_sc_segment(compute_cnt):
    mesh = plsc.VectorSubcoreMesh(core_axis_name="c", subcore_axis_name="s")
    out_type = [jax.ShapeDtypeStruct((NC, NP, H), jnp.float32)]
    scratch = [
        pltpu.VMEM((CPW, CHUNK), jnp.int32),
        pltpu.VMEM((CPW, CHUNK), jnp.int32),
        pltpu.VMEM((NBUF, CHUNK, H), jnp.float32),
    ]
    if compute_cnt:
        out_type.append(jax.ShapeDtypeStruct((NC, NP), jnp.float32))
        scratch.append(pltpu.VMEM((CHUNK,), jnp.float32))
    scratch.append(pltpu.VMEM((CHUNK, H), jnp.float32))
    if compute_cnt:
        scratch.append(pltpu.VMEM((CHUNK,), jnp.float32))
    scratch.append(pltpu.VMEM_SHARED((NP, H), jnp.float32))
    if compute_cnt:
        scratch.append(pltpu.VMEM_SHARED((NP,), jnp.float32))
    scratch.append(pltpu.SemaphoreType.DMA((NBUF,)))
    scratch.append(pltpu.SemaphoreType.DMA((NBUF,)))
    if compute_cnt:
        scratch.append(pltpu.SemaphoreType.DMA((2,)))
    return pl.kernel(
        functools.partial(_seg_body, compute_cnt),
        out_type=tuple(out_type),
        mesh=mesh,
        scratch_types=tuple(scratch),
        compiler_params=pltpu.CompilerParams(use_tc_tiling_on_sc=False),
    )


def _tc_pre(x_p, W1l, W1r):
    def body(x_ref, wl_ref, wr_ref, y_ref, xr_ref):
        xb = x_ref[...]
        dn = (((1,), (1,)), ((), ()))
        y_ref[...] = lax.dot_general(xb, wl_ref[...], dn,
                                     preferred_element_type=jnp.float32)
        xr_ref[...] = lax.dot_general(xb, wr_ref[...], dn,
                                      preferred_element_type=jnp.float32)
    return pl.pallas_call(
        body,
        out_shape=(jax.ShapeDtypeStruct((NP, H), jnp.float32),
                   jax.ShapeDtypeStruct((NP, H), jnp.float32)),
    )(x_p, W1l, W1r)


def _tc_mid(part, cntp, xr1, b1, W2l, W2r):
    def body(part_ref, cnt_ref, xr_ref, b1_ref, wl_ref, wr_ref,
             y2_ref, hr2_ref):
        seg = part_ref[0] + part_ref[1]
        cnt = jnp.clip(cnt_ref[0] + cnt_ref[1], 1.0, None)
        h = jax.nn.sigmoid(seg / cnt + b1_ref[...] + xr_ref[...])
        dn = (((1,), (1,)), ((), ()))
        y2_ref[...] = lax.dot_general(h, wl_ref[...], dn,
                                      preferred_element_type=jnp.float32)
        hr2_ref[...] = lax.dot_general(h, wr_ref[...], dn,
                                       preferred_element_type=jnp.float32)
    return pl.pallas_call(
        body,
        out_shape=(jax.ShapeDtypeStruct((NP, H), jnp.float32),
                   jax.ShapeDtypeStruct((NP, H), jnp.float32)),
    )(part, cntp, xr1, b1, W2l, W2r)


def _tc_final(part, cntp, hr2, b2):
    def body(part_ref, cnt_ref, hr_ref, b2_ref, out_ref):
        seg = part_ref[0] + part_ref[1]
        cnt = jnp.clip(cnt_ref[0] + cnt_ref[1], 1.0, None)
        z = seg / cnt + b2_ref[...] + hr_ref[...]
        m = jnp.max(z, axis=1, keepdims=True)
        lse = jnp.log(jnp.sum(jnp.exp(z - m), axis=1, keepdims=True)) + m
        out_ref[...] = z - lse
    return pl.pallas_call(
        body,
        out_shape=jax.ShapeDtypeStruct((NP, 16), jnp.float32),
    )(part, cntp, hr2, b2)


@jax.jit
def kernel(x, edge_index, W1l, b1, W1r, W2l, b2, W2r):
    src = edge_index[0].astype(jnp.int32)
    dst = edge_index[1].astype(jnp.int32)
    pad = EP - E
    src_p = jnp.concatenate([src, jnp.zeros((pad,), jnp.int32)])
    dst_p = jnp.concatenate([dst, jnp.full((pad,), N, jnp.int32)])
    src_p = src_p.reshape(NW, CPW, CHUNK)
    dst_p = dst_p.reshape(NW, CPW, CHUNK)
    x_p = jnp.pad(x, ((0, NP - N), (0, 0)))

    y1, xr1 = _tc_pre(x_p, W1l, W1r)
    part1, cnt = _make_sc_segment(True)(src_p, dst_p, y1)
    cntp = cnt.reshape(NC, NP, 1)
    y2, hr2 = _tc_mid(part1, cntp, xr1, b1.reshape(1, H), W2l, W2r)
    (part2,) = _make_sc_segment(False)(src_p, dst_p, y2)
    out = _tc_final(part2, cntp, hr2, b2.reshape(1, 16))
    return out[:N]


# speedup vs baseline: 14.8258x; 1.0174x over previous
import functools

import jax
import jax.numpy as jnp
from jax import lax
from jax.experimental import pallas as pl
from jax.experimental.pallas import tpu as pltpu
from jax.experimental.pallas import tpu_sc as plsc

N = 10000
D = 128
H = 16
E = 320000

NC = 2
NS = 16
NW = NC * NS
CHUNK = 128
CPW = 80
EPW = CPW * CHUNK
EP = NW * EPW
NBUF = 8
GA = NBUF // 2

NP = 10240
RPS = NP // NS


def _seg_body(compute_cnt, src_hbm, dst_hbm, y_hbm, *rest):
    if compute_cnt:
        part_out, cnt_out, src_v, dst_v, rows_v, ones_v, zrow_v, zcnt_v, \
            acc_sh, cnt_sh, gsem, ssem, csem = rest
    else:
        part_out, src_v, dst_v, rows_v, zrow_v, acc_sh, gsem, ssem = rest

    c = lax.axis_index("c")
    s = lax.axis_index("s")
    wid = s * NC + c

    idesc = [pltpu.async_copy(src_hbm.at[wid], src_v, gsem.at[0]),
             pltpu.async_copy(dst_hbm.at[wid], dst_v, gsem.at[1])]

    def fill_zrow(i, _):
        zrow_v[i] = jnp.zeros((16,), jnp.float32)
        return 0
    lax.fori_loop(0, CHUNK, fill_zrow, 0)
    if compute_cnt:
        def fill_ones(i, _):
            ones_v[pl.ds(i * 16, 16)] = jnp.ones((16,), jnp.float32)
            zcnt_v[pl.ds(i * 16, 16)] = jnp.zeros((16,), jnp.float32)
            return 0
        lax.fori_loop(0, CHUNK // 16, fill_ones, 0)

    for k in range(RPS // CHUNK):
        pltpu.async_copy(zrow_v, acc_sh.at[pl.ds(s * RPS + k * CHUNK, CHUNK)],
                         ssem.at[k])
        if compute_cnt:
            pltpu.async_copy(
                zcnt_v, cnt_sh.at[pl.ds(s * RPS + k * CHUNK, CHUNK)],
                csem.at[k % 8])
    for k in range(RPS // CHUNK):
        pltpu.make_async_copy(
            zrow_v, acc_sh.at[pl.ds(s * RPS + k * CHUNK, CHUNK)],
            ssem.at[k]).wait()
        if compute_cnt:
            pltpu.make_async_copy(
                zcnt_v, cnt_sh.at[pl.ds(s * RPS + k * CHUNK, CHUNK)],
                csem.at[k % 8]).wait()
    for d in idesc:
        d.wait()
    plsc.subcore_barrier()

    def gather(j, b):
        pltpu.async_copy(y_hbm.at[src_v.at[j]], rows_v.at[b], gsem.at[b])

    for b in range(GA):
        gather(b, b)

    def group(g, _):
        for b in range(NBUF):
            j = g * NBUF + b
            pltpu.make_async_copy(
                y_hbm.at[src_v.at[j]], rows_v.at[b], gsem.at[b]).wait()
            pltpu.async_copy(
                rows_v.at[b], acc_sh.at[dst_v.at[j]], ssem.at[b], add=True)
            if compute_cnt:
                cb = b % 8

                @pl.when(j >= 8)
                def _():
                    pltpu.make_async_copy(
                        ones_v, cnt_sh.at[dst_v.at[j - 8]],
                        csem.at[cb]).wait()
                pltpu.async_copy(
                    ones_v, cnt_sh.at[dst_v.at[j]], csem.at[cb], add=True)

            nb = (b + GA) % NBUF

            @pl.when(j + GA < CPW)
            def _():
                @pl.when(j >= NBUF - GA)
                def _():
                    pltpu.make_async_copy(
                        rows_v.at[nb], acc_sh.at[dst_v.at[j - (NBUF - GA)]],
                        ssem.at[nb]).wait()
                gather(j + GA, nb)
        return 0
    lax.fori_loop(0, CPW // NBUF, group, 0)

    for b in range(NBUF):
        pltpu.make_async_copy(
            rows_v.at[b], acc_sh.at[dst_v.at[CPW - NBUF + b]],
            ssem.at[b]).wait()
    if compute_cnt:
        for cb in range(8):
            pltpu.make_async_copy(
                ones_v, cnt_sh.at[dst_v.at[CPW - 8 + cb]],
                csem.at[cb]).wait()
    plsc.subcore_barrier()

    pltpu.sync_copy(acc_sh.at[pl.ds(s * RPS, RPS)],
                    part_out.at[c, pl.ds(s * RPS, RPS)])
    if compute_cnt:
        pltpu.sync_copy(cnt_sh.at[pl.ds(s * RPS, RPS)],
                        cnt_out.at[c, pl.ds(s * RPS, RPS)])


def _make_sc_segment(compute_cnt):
    mesh = plsc.VectorSubcoreMesh(core_axis_name="c", subcore_axis_name="s")
    out_type = [jax.ShapeDtypeStruct((NC, NP, H), jnp.float32)]
    scratch = [
        pltpu.VMEM((CPW, CHUNK), jnp.int32),
        pltpu.VMEM((CPW, CHUNK), jnp.int32),
        pltpu.VMEM((NBUF, CHUNK, H), jnp.float32),
    ]
    if compute_cnt:
        out_type.append(jax.ShapeDtypeStruct((NC, NP), jnp.float32))
        scratch.append(pltpu.VMEM((CHUNK,), jnp.float32))
    scratch.append(pltpu.VMEM((CHUNK, H), jnp.float32))
    if compute_cnt:
        scratch.append(pltpu.VMEM((CHUNK,), jnp.float32))
    scratch.append(pltpu.VMEM_SHARED((NP, H), jnp.float32))
    if compute_cnt:
        scratch.append(pltpu.VMEM_SHARED((NP,), jnp.float32))
    scratch.append(pltpu.SemaphoreType.DMA((NBUF,)))
    scratch.append(pltpu.SemaphoreType.DMA((NBUF,)))
    if compute_cnt:
        scratch.append(pltpu.SemaphoreType.DMA((8,)))
    return pl.kernel(
        functools.partial(_seg_body, compute_cnt),
        out_type=tuple(out_type),
        mesh=mesh,
        scratch_types=tuple(scratch),
        compiler_params=pltpu.CompilerParams(use_tc_tiling_on_sc=False),
    )


def _tc_pre(x_p, W1l, W1r):
    def body(x_ref, wl_ref, wr_ref, y_ref, xr_ref):
        xb = x_ref[...]
        dn = (((1,), (1,)), ((), ()))
        y_ref[...] = lax.dot_general(xb, wl_ref[...], dn,
                                     preferred_element_type=jnp.float32)
        xr_ref[...] = lax.dot_general(xb, wr_ref[...], dn,
                                      preferred_element_type=jnp.float32)
    return pl.pallas_call(
        body,
        out_shape=(jax.ShapeDtypeStruct((NP, H), jnp.float32),
                   jax.ShapeDtypeStruct((NP, H), jnp.float32)),
    )(x_p, W1l, W1r)


def _tc_mid(part, cntp, xr1, b1, W2l, W2r):
    def body(part_ref, cnt_ref, xr_ref, b1_ref, wl_ref, wr_ref,
             y2_ref, hr2_ref):
        seg = part_ref[0] + part_ref[1]
        cnt = jnp.clip(cnt_ref[0] + cnt_ref[1], 1.0, None)
        h = jax.nn.sigmoid(seg / cnt + b1_ref[...] + xr_ref[...])
        dn = (((1,), (1,)), ((), ()))
        y2_ref[...] = lax.dot_general(h, wl_ref[...], dn,
                                      preferred_element_type=jnp.float32)
        hr2_ref[...] = lax.dot_general(h, wr_ref[...], dn,
                                       preferred_element_type=jnp.float32)
    return pl.pallas_call(
        body,
        out_shape=(jax.ShapeDtypeStruct((NP, H), jnp.float32),
                   jax.ShapeDtypeStruct((NP, H), jnp.float32)),
    )(part, cntp, xr1, b1, W2l, W2r)


def _tc_final(part, cntp, hr2, b2):
    def body(part_ref, cnt_ref, hr_ref, b2_ref, out_ref):
        seg = part_ref[0] + part_ref[1]
        cnt = jnp.clip(cnt_ref[0] + cnt_ref[1], 1.0, None)
        z = seg / cnt + b2_ref[...] + hr_ref[...]
        m = jnp.max(z, axis=1, keepdims=True)
        lse = jnp.log(jnp.sum(jnp.exp(z - m), axis=1, keepdims=True)) + m
        out_ref[...] = z - lse
    return pl.pallas_call(
        body,
        out_shape=jax.ShapeDtypeStruct((NP, 16), jnp.float32),
    )(part, cntp, hr2, b2)


@jax.jit
def kernel(x, edge_index, W1l, b1, W1r, W2l, b2, W2r):
    src = edge_index[0].astype(jnp.int32)
    dst = edge_index[1].astype(jnp.int32)
    pad = EP - E
    src_p = jnp.concatenate([src, jnp.zeros((pad,), jnp.int32)])
    dst_p = jnp.concatenate([dst, jnp.full((pad,), N, jnp.int32)])
    src_p = src_p.reshape(NW, CPW, CHUNK)
    dst_p = dst_p.reshape(NW, CPW, CHUNK)
    x_p = jnp.pad(x, ((0, NP - N), (0, 0)))

    y1, xr1 = _tc_pre(x_p, W1l, W1r)
    part1, cnt = _make_sc_segment(True)(src_p, dst_p, y1)
    cntp = cnt.reshape(NC, NP, 1)
    y2, hr2 = _tc_mid(part1, cntp, xr1, b1.reshape(1, H), W2l, W2r)
    (part2,) = _make_sc_segment(False)(src_p, dst_p, y2)
    out = _tc_final(part2, cntp, hr2, b2.reshape(1, 16))
    return out[:N]
